# baseline (speedup 1.0000x reference)
import functools

import jax
import jax.numpy as jnp
from jax import lax
from jax.experimental import pallas as pl
from jax.experimental.pallas import tpu as pltpu
from jax.experimental.pallas import tpu_sc as plsc

VOCAB = 1000000
DIM = 64
B = 4096
N = 200
NC, NS = 2, 16
NW = NC * NS
NGRP = N // 8
NCB = B // 128
NTASK = NGRP * NCB
PER_W = NTASK // NW

_mesh = plsc.VectorSubcoreMesh(core_axis_name="c", subcore_axis_name="s")


@functools.partial(
    pl.kernel,
    out_type=jax.ShapeDtypeStruct((N, DIM, B), jnp.float32),
    mesh=_mesh,
    compiler_params=pltpu.CompilerParams(
        use_tc_tiling_on_sc=True,
        needs_layout_passes=False,
        disable_bounds_checks=True,
    ),
    scratch_types=[
        pltpu.VMEM((8, 128), jnp.int32),
        pltpu.VMEM((8, 128), jnp.int32),
        pltpu.VMEM((8, 128), jnp.int32),
        pltpu.VMEM((128, 128), jnp.float32),
        pltpu.VMEM((128, 128), jnp.float32),
        pltpu.VMEM((DIM, 128), jnp.float32),
        pltpu.VMEM((DIM, 128), jnp.float32),
        pltpu.SemaphoreType.DMA,
        pltpu.SemaphoreType.DMA,
        pltpu.SemaphoreType.DMA,
        pltpu.SemaphoreType.DMA,
        pltpu.SemaphoreType.DMA,
    ],
)
def _emb_lookup(idsT_hbm, tab_hbm, out_hbm, idt_v, pidx_v, hoff_v, rows0, rows1, t0, t1,
                isem, gsem0, gsem1, wsem0, wsem1):
    wid = lax.axis_index("s") * NC + lax.axis_index("c")
    rows = (rows0, rows1)
    tbufs = (t0, t1)
    gsems = (gsem0, gsem1)
    wsems = (wsem0, wsem1)


    rvs = tuple(lax.iota(jnp.int32, 16) + j0 for j0 in range(0, 128, 16))

    def start_gather(r, b):
        pltpu.async_copy(tab_hbm.at[pidx_v.at[r]], rows[b], gsems[b])

    def wait_gather(r, b):
        pltpu.make_async_copy(tab_hbm.at[pidx_v.at[r]], rows[b],
                              gsems[b]).wait()

    def transpose_block(r, b):
        tb = tbufs[b]
        rb = rows[b]
        hvs = tuple(hoff_v[r, pl.ds(j0 * 16, 16)] for j0 in range(8))

        def dbody(d0, carry):
            for dd in range(8):
                d = d0 * 8 + dd
                for j0 in range(8):
                    vec = plsc.load_gather(rb, [rvs[j0], hvs[j0] + d])
                    tb[d, pl.ds(j0 * 16, 16)] = vec
            return carry

        lax.fori_loop(0, DIM // 8, dbody, 0)

    def start_write(g, c, r, b):
        pltpu.async_copy(
            tbufs[b],
            out_hbm.at[g * 8 + r].at[:, pl.ds(c * 128, 128)],
            wsems[b])

    def wait_write(g, c, r, b):
        pltpu.make_async_copy(
            tbufs[b],
            out_hbm.at[g * 8 + r].at[:, pl.ds(c * 128, 128)],
            wsems[b]).wait()

    def run(t, carry):
        g = t // NCB
        c = t % NCB
        pltpu.async_copy(
            idsT_hbm.at[pl.ds(g * 8, 8), pl.ds(c * 128, 128)], idt_v,
            isem).wait()
        for r in range(8):
            for j0 in range(0, 128, 16):
                v = idt_v[r, pl.ds(j0, 16)]
                pidx_v[r, pl.ds(j0, 16)] = v >> 1
                hoff_v[r, pl.ds(j0, 16)] = (v & 1) * DIM
        start_gather(0, 0)
        for r in range(8):
            b = r % 2
            if r + 1 < 8:
                start_gather(r + 1, 1 - b)
            wait_gather(r, b)
            if r >= 2:
                wait_write(g, c, r - 2, b)
            transpose_block(r, b)
            start_write(g, c, r, b)
        wait_write(g, c, 6, 0)
        wait_write(g, c, 7, 1)
        return carry

    lax.fori_loop(wid * PER_W, (wid + 1) * PER_W, run, 0)


def kernel(input_ids, table):
    tab2 = table.reshape(VOCAB // 2, 2 * DIM)
    ids_t = input_ids.T
    out_t = _emb_lookup(ids_t, tab2)
    return jnp.transpose(out_t, (2, 0, 1))


# speedup vs baseline: 1.4367x; 1.4367x over previous
import functools

import jax
import jax.numpy as jnp
from jax import lax
from jax.experimental import pallas as pl
from jax.experimental.pallas import tpu as pltpu
from jax.experimental.pallas import tpu_sc as plsc

VOCAB = 1000000
DIM = 64
B = 4096
N = 200
NC, NS = 2, 16
NW = NC * NS
NGRP = N // 8
NCB = B // 128
NTASK = NGRP * NCB
PER_W = NTASK // NW

_mesh = plsc.VectorSubcoreMesh(core_axis_name="c", subcore_axis_name="s")


@functools.partial(
    pl.kernel,
    out_type=jax.ShapeDtypeStruct((N, DIM, B), jnp.float32),
    mesh=_mesh,
    compiler_params=pltpu.CompilerParams(
        use_tc_tiling_on_sc=True,
        needs_layout_passes=False,
        disable_bounds_checks=True,
    ),
    scratch_types=[
        pltpu.VMEM((8, 128), jnp.int32),
        pltpu.VMEM((8, 128), jnp.int32),
        pltpu.VMEM((8, 128), jnp.int32),
        pltpu.VMEM((128, 128), jnp.float32),
        pltpu.VMEM((128, 128), jnp.float32),
        pltpu.VMEM((DIM, 128), jnp.float32),
        pltpu.VMEM((DIM, 128), jnp.float32),
        pltpu.SemaphoreType.DMA,
        pltpu.SemaphoreType.DMA,
        pltpu.SemaphoreType.DMA,
        pltpu.SemaphoreType.DMA,
        pltpu.SemaphoreType.DMA,
    ],
)
def _emb_lookup(idsT_hbm, tab_hbm, out_hbm, idt_v, pidx_v, hoff_v, rows0, rows1, t0, t1,
                isem, gsem0, gsem1, wsem0, wsem1):
    wid = lax.axis_index("s") * NC + lax.axis_index("c")
    rows = (rows0, rows1)
    tbufs = (t0, t1)
    gsems = (gsem0, gsem1)
    wsems = (wsem0, wsem1)


    rvs = tuple(lax.iota(jnp.int32, 16) + j0 for j0 in range(0, 128, 16))

    def start_gather(r, b):
        pltpu.async_copy(tab_hbm.at[pidx_v.at[r]], rows[b], gsems[b])

    def wait_gather(r, b):
        pltpu.make_async_copy(tab_hbm.at[pidx_v.at[r]], rows[b],
                              gsems[b]).wait()

    def transpose_block(r, b):
        tb = tbufs[b]
        rb = rows[b]
        hvs = tuple(hoff_v[r, pl.ds(j0 * 16, 16)] for j0 in range(8))

        @plsc.parallel_loop(0, DIM, step=1, unroll=8)
        def dbody(d):
            for j0 in range(8):
                vec = plsc.load_gather(rb, [rvs[j0], hvs[j0] + d])
                tb[d, pl.ds(j0 * 16, 16)] = vec

    def start_write(g, c, r, b):
        pltpu.async_copy(
            tbufs[b],
            out_hbm.at[g * 8 + r].at[:, pl.ds(c * 128, 128)],
            wsems[b])

    def wait_write(g, c, r, b):
        pltpu.make_async_copy(
            tbufs[b],
            out_hbm.at[g * 8 + r].at[:, pl.ds(c * 128, 128)],
            wsems[b]).wait()

    def run(t, carry):
        g = t // NCB
        c = t % NCB
        pltpu.async_copy(
            idsT_hbm.at[pl.ds(g * 8, 8), pl.ds(c * 128, 128)], idt_v,
            isem).wait()
        for r in range(8):
            for j0 in range(0, 128, 16):
                v = idt_v[r, pl.ds(j0, 16)]
                pidx_v[r, pl.ds(j0, 16)] = v >> 1
                hoff_v[r, pl.ds(j0, 16)] = (v & 1) * DIM
        start_gather(0, 0)
        for r in range(8):
            b = r % 2
            if r + 1 < 8:
                start_gather(r + 1, 1 - b)
            wait_gather(r, b)
            if r >= 2:
                wait_write(g, c, r - 2, b)
            transpose_block(r, b)
            start_write(g, c, r, b)
        wait_write(g, c, 6, 0)
        wait_write(g, c, 7, 1)
        return carry

    lax.fori_loop(wid * PER_W, (wid + 1) * PER_W, run, 0)


def kernel(input_ids, table):
    tab2 = table.reshape(VOCAB // 2, 2 * DIM)
    ids_t = input_ids.T
    out_t = _emb_lookup(ids_t, tab2)
    return jnp.transpose(out_t, (2, 0, 1))
